# baseline (speedup 1.0000x reference)
import jax
import jax.numpy as jnp
from jax import lax
from jax.experimental import pallas as pl
from jax.experimental.pallas import tpu as pltpu
from jax.experimental.pallas import tpu_sc as plsc

NUM_FACTORS = 32
NPACK = NUM_FACTORS // 2
MU = 3.5
BATCH = 16384
NC = 2
NS = 16
L = 16
NW = NC * NS
BPW = BATCH // NW
CHUNK = 128
NCHUNK = BPW // CHUNK


def _sc_body(uidx_hbm, iidx_hbm, uw_hbm, iw_hbm, ub_hbm, ib_hbm,
             urows_hbm, irows_hbm, bsum_hbm,
             uidx_v, iidx_v, urows_v, irows_v, ub_v, ib_v, sem):
    c = lax.axis_index("c")
    s = lax.axis_index("s")
    wid = s * NC + c

    pltpu.sync_copy(uidx_hbm.at[wid], uidx_v)
    pltpu.sync_copy(iidx_hbm.at[wid], iidx_v)

    copies = []
    for j in range(NCHUNK):
        dst = pl.ds(j * CHUNK, CHUNK)
        copies.append(pltpu.async_copy(uw_hbm.at[uidx_v.at[j]], urows_v.at[dst], sem))
        copies.append(pltpu.async_copy(iw_hbm.at[iidx_v.at[j]], irows_v.at[dst], sem))
        copies.append(pltpu.async_copy(ub_hbm.at[uidx_v.at[j]], ub_v.at[dst], sem))
        copies.append(pltpu.async_copy(ib_hbm.at[iidx_v.at[j]], ib_v.at[dst], sem))
    for cp in copies:
        cp.wait()

    def addb(g, carry):
        sl = pl.ds(g * L, L)
        ub_v[sl] = ub_v[sl] + ib_v[sl]
        return carry

    lax.fori_loop(0, BPW // L, addb, 0)

    base = pl.ds(wid * BPW, BPW)
    pltpu.sync_copy(urows_v, urows_hbm.at[base])
    pltpu.sync_copy(irows_v, irows_hbm.at[base])
    pltpu.sync_copy(ub_v, bsum_hbm.at[base])


@jax.jit
def _run(uidx3, iidx3, uw, iw, ubf, ibf):
    mesh = plsc.VectorSubcoreMesh(core_axis_name="c", subcore_axis_name="s")
    f = pl.kernel(
        _sc_body,
        mesh=mesh,
        compiler_params=pltpu.CompilerParams(use_tc_tiling_on_sc=False),
        out_type=(jax.ShapeDtypeStruct((BATCH, NPACK), jnp.int32),
                  jax.ShapeDtypeStruct((BATCH, NPACK), jnp.int32),
                  jax.ShapeDtypeStruct((BATCH,), jnp.float32)),
        scratch_types=[
            pltpu.VMEM((NCHUNK, CHUNK), jnp.int32),
            pltpu.VMEM((NCHUNK, CHUNK), jnp.int32),
            pltpu.VMEM((BPW, NPACK), jnp.int32),
            pltpu.VMEM((BPW, NPACK), jnp.int32),
            pltpu.VMEM((BPW,), jnp.float32),
            pltpu.VMEM((BPW,), jnp.float32),
            pltpu.SemaphoreType.DMA,
        ],
    )
    return f(uidx3, iidx3, uw, iw, ubf, ibf)


def _tc_dot_body(uw_ref, iw_ref, bs_ref, out_ref):
    uw = uw_ref[...]
    iw = iw_ref[...]
    himask = jnp.int32(-65536)
    ua = lax.bitcast_convert_type(lax.shift_left(uw, 16), jnp.float32)
    ub2 = lax.bitcast_convert_type(uw & himask, jnp.float32)
    ia = lax.bitcast_convert_type(lax.shift_left(iw, 16), jnp.float32)
    ib2 = lax.bitcast_convert_type(iw & himask, jnp.float32)
    dots = jnp.sum(ua * ia + ub2 * ib2, axis=1)
    out_ref[...] = dots + bs_ref[...] + MU


def _tc_dot(urows, irows, bsum):
    blk = 2048
    return pl.pallas_call(
        _tc_dot_body,
        grid=(BATCH // blk,),
        in_specs=[
            pl.BlockSpec((blk, NPACK), lambda i: (i, 0)),
            pl.BlockSpec((blk, NPACK), lambda i: (i, 0)),
            pl.BlockSpec((blk,), lambda i: (i,)),
        ],
        out_specs=pl.BlockSpec((blk,), lambda i: (i,)),
        out_shape=jax.ShapeDtypeStruct((BATCH,), jnp.float32),
    )(urows, irows, bsum)


def kernel(user_idx, item_idx, embed_user_w, embed_item_w, user_bias_w, item_bias_w):
    uidx3 = user_idx.reshape(NW, NCHUNK, CHUNK)
    iidx3 = item_idx.reshape(NW, NCHUNK, CHUNK)
    ubf = user_bias_w.reshape(-1)
    ibf = item_bias_w.reshape(-1)
    uwp = jax.lax.bitcast_convert_type(
        embed_user_w.astype(jnp.bfloat16).reshape(-1, NPACK, 2), jnp.int32)
    iwp = jax.lax.bitcast_convert_type(
        embed_item_w.astype(jnp.bfloat16).reshape(-1, NPACK, 2), jnp.int32)
    urows, irows, bsum = _run(uidx3, iidx3, uwp, iwp, ubf, ibf)
    return _tc_dot(urows, irows, bsum)
